# baseline (speedup 1.0000x reference)
import functools

import jax
import jax.numpy as jnp
from jax import lax
from jax.experimental import pallas as pl
from jax.experimental.pallas import tpu as pltpu
from jax.experimental.pallas import tpu_sc as plsc

NUM_RELATIONS = 1000000
EMBEDDING_DIM = 32
BATCH = 4096
FIELDS = 26

_info = plsc.get_sparse_core_info()
_NC, _NS = _info.num_cores, _info.num_subcores
_NW = _NC * _NS
_NROWS = FIELDS * EMBEDDING_DIM
_RPW = _NROWS // _NW
_NCH = 2 * _RPW
_HB = BATCH // 2
_NBUF = 2


@functools.partial(
    pl.kernel,
    out_type=jax.ShapeDtypeStruct((FIELDS, EMBEDDING_DIM, BATCH), jnp.float32),
    mesh=plsc.VectorSubcoreMesh(core_axis_name="c", subcore_axis_name="s"),
    scratch_types=[
        pltpu.VMEM((_NBUF, _HB), jnp.int32),
        pltpu.VMEM((_NBUF, _HB, 16), jnp.float32),
        pltpu.VMEM((_NBUF, _HB), jnp.float32),
        pltpu.SemaphoreType.DMA((_NBUF,)),
        pltpu.SemaphoreType.DMA((_NBUF,)),
        pltpu.SemaphoreType.DMA((_NBUF,)),
    ],
    compiler_params=pltpu.CompilerParams(use_tc_tiling_on_sc=False),
)
def _gather_kernel(tab_hbm, ids_hbm, out_hbm, idx_v, stg_v, row_v, isems, gsems, wsems):
    wid = lax.axis_index("s") * _NC + lax.axis_index("c")
    r0 = wid * _RPW

    ih = [None] * _NCH
    gh = [None] * _NCH
    wh = [None] * _NCH

    def row_fd(k):
        r = r0 + k // 2
        return r // EMBEDDING_DIM, r % EMBEDDING_DIM, (k % 2) * _HB

    for t in range(_NCH + 2):
        k = t - 2
        if 0 <= k < _NCH:
            b = k % _NBUF
            gh[k].wait()
            f, d, h = row_fd(k)
            wh[k] = pltpu.async_copy(
                row_v.at[b], out_hbm.at[f, d, pl.ds(h, _HB)], wsems.at[b]
            )
        k = t - 1
        if 0 <= k < _NCH:
            b = k % _NBUF
            ih[k].wait()
            gh[k] = pltpu.async_copy(
                tab_hbm.at[idx_v.at[b]], stg_v.at[b], gsems.at[b]
            )
        k = t
        if k < _NCH:
            b = k % _NBUF
            if k >= _NBUF:
                wh[k - _NBUF].wait()
            f, _, h = row_fd(k)
            ih[k] = pltpu.async_copy(
                ids_hbm.at[f, pl.ds(h, _HB)], idx_v.at[b], isems.at[b]
            )
    for k in range(_NCH - _NBUF, _NCH):
        wh[k].wait()


@jax.jit
def kernel(relation_ids, embedding_table):
    tab4 = embedding_table.T.reshape(2_000_000, 16)
    ids_t = relation_ids.T.astype(jnp.int32)
    out = _gather_kernel(tab4, ids_t)
    return out.transpose(2, 0, 1)


# speedup vs baseline: 1.0118x; 1.0118x over previous
import functools

import jax
import jax.numpy as jnp
from jax import lax
from jax.experimental import pallas as pl
from jax.experimental.pallas import tpu as pltpu
from jax.experimental.pallas import tpu_sc as plsc

NUM_RELATIONS = 1000000
EMBEDDING_DIM = 32
BATCH = 4096
FIELDS = 26

_info = plsc.get_sparse_core_info()
_NC, _NS = _info.num_cores, _info.num_subcores
_NW = _NC * _NS
_NROWS = FIELDS * EMBEDDING_DIM
_RPW = _NROWS // _NW
_CS = 512
_CPR = BATCH // _CS
_NCH = _RPW * _CPR
_RING = 8
_GLAG = 2
_DLAG = 6
_NG = _CS // 16


@functools.partial(
    pl.kernel,
    out_type=jax.ShapeDtypeStruct((FIELDS, EMBEDDING_DIM, BATCH), jnp.float32),
    mesh=plsc.VectorSubcoreMesh(core_axis_name="c", subcore_axis_name="s"),
    scratch_types=[
        pltpu.VMEM((_RING, _CS), jnp.int32),
        pltpu.VMEM((_RING, _CS), jnp.int32),
        pltpu.VMEM((_RING, _CS, 16), jnp.float32),
        pltpu.VMEM((_RING, _CS), jnp.float32),
        pltpu.SemaphoreType.DMA((_RING,)),
        pltpu.SemaphoreType.DMA((_RING,)),
        pltpu.SemaphoreType.DMA((_RING,)),
    ],
    compiler_params=pltpu.CompilerParams(
        use_tc_tiling_on_sc=False, needs_layout_passes=False
    ),
)
def _gather_kernel(
    tab_hbm, ids_hbm, out_hbm, idx_v, bidx_v, stg_v, row_v, isems, gsems, wsems
):
    wid = lax.axis_index("s") * _NC + lax.axis_index("c")
    r0 = wid * _RPW
    lane = lax.iota(jnp.int32, 16)

    def chunk_fdq(cc):
        r = r0 + cc // _CPR
        return r // EMBEDDING_DIM, r % EMBEDDING_DIM, cc % _CPR

    def body(t, _):
        @pl.when(t < _NCH)
        def _():
            b = t % _RING

            @pl.when(t >= _RING)
            def _():
                pltpu.make_async_copy(
                    row_v.at[b], out_hbm.at[0, 0, pl.ds(0, _CS)], wsems.at[b]
                ).wait()

            f, _unused, q = chunk_fdq(t)
            pltpu.async_copy(
                ids_hbm.at[f, pl.ds(q * _CS, _CS)], idx_v.at[b], isems.at[b]
            )

        @pl.when(jnp.logical_and(t >= _GLAG, t < _NCH + _GLAG))
        def _():
            cc = t - _GLAG
            b = cc % _RING
            pltpu.make_async_copy(
                ids_hbm.at[0, pl.ds(0, _CS)], idx_v.at[b], isems.at[b]
            ).wait()
            _unused_f, d, _unused_q = chunk_fdq(cc)
            doff = d * (NUM_RELATIONS // 16)
            for g in range(_NG):
                sl = pl.ds(g * 16, 16)
                bidx_v[b, sl] = lax.shift_right_logical(idx_v[b, sl], 4) + doff
            pltpu.async_copy(tab_hbm.at[bidx_v.at[b]], stg_v.at[b], gsems.at[b])

        @pl.when(jnp.logical_and(t >= _DLAG, t < _NCH + _DLAG))
        def _():
            cc = t - _DLAG
            b = cc % _RING
            pltpu.make_async_copy(
                tab_hbm.at[bidx_v.at[b]], stg_v.at[b], gsems.at[b]
            ).wait()
            for g in range(_NG):
                sl = pl.ds(g * 16, 16)
                low = lax.bitwise_and(idx_v[b, sl], 15)
                row16 = plsc.load_gather(stg_v.at[b], [g * 16 + lane, low])
                row_v[b, sl] = row16
            f, d, q = chunk_fdq(cc)
            pltpu.async_copy(
                row_v.at[b], out_hbm.at[f, d, pl.ds(q * _CS, _CS)], wsems.at[b]
            )

        return ()

    lax.fori_loop(0, _NCH + _DLAG, body, (), unroll=False)
    for b in range(_RING):
        pltpu.make_async_copy(
            row_v.at[b], out_hbm.at[0, 0, pl.ds(0, _CS)], wsems.at[b]
        ).wait()


@jax.jit
def kernel(relation_ids, embedding_table):
    tab4 = embedding_table.T.reshape(EMBEDDING_DIM * NUM_RELATIONS // 16, 16)
    ids_t = relation_ids.T.astype(jnp.int32)
    out = _gather_kernel(tab4, ids_t)
    return out.transpose(2, 0, 1)
